# baseline (speedup 1.0000x reference)
import functools

import jax
import jax.numpy as jnp
from jax import lax
from jax.experimental import pallas as pl
from jax.experimental.pallas import tpu as pltpu
from jax.experimental.pallas import tpu_sc as plsc

BATCH = 512
NUM_NEURONS = 8192
HISTORY_LEN = 8
_NL = 128
_NC = NUM_NEURONS // _NL
_NW = 32
_RPW = BATCH // _NW


def kernel(x, state):
    st4 = state.reshape(BATCH, _NC, _NL, HISTORY_LEN).transpose(0, 1, 3, 2)
    x4 = x.reshape(BATCH // 8, 8, _NC, _NL).transpose(0, 2, 1, 3)

    mesh = plsc.VectorSubcoreMesh(core_axis_name="c", subcore_axis_name="s")

    _NBUF = 3
    _UPR = 2
    _HC = _NC // _UPR
    _NU = _RPW * _UPR

    @functools.partial(
        pl.kernel,
        mesh=mesh,
        out_type=jax.ShapeDtypeStruct((BATCH, _NC, HISTORY_LEN, _NL), jnp.float32),
        scratch_types=[
            pltpu.VMEM((_NBUF, _HC, HISTORY_LEN, _NL), jnp.float32),
            pltpu.SemaphoreType.DMA((_NBUF,)),
            pltpu.SemaphoreType.DMA((_NBUF,)),
            pltpu.SemaphoreType.DMA((_NBUF,)),
            pltpu.SemaphoreType.DMA((_NBUF,)),
        ],
    )
    def push(x4_hbm, st4_hbm, out4_hbm, buf, sgs, sgx, sss, ss2):
        wid = lax.axis_index("s") * 2 + lax.axis_index("c")
        base = wid * _RPW

        gath = [None] * _NBUF
        scat = [None] * _NBUF

        def start_gather(u):
            b = base + u // _UPR
            c0 = (u % _UPR) * _HC
            j = u % _NBUF
            g1 = pltpu.async_copy(
                st4_hbm.at[b, c0 : c0 + _HC, 0 : HISTORY_LEN - 1, :],
                buf.at[j, :, 1:HISTORY_LEN, :],
                sgs.at[j],
            )
            g2 = pltpu.async_copy(
                x4_hbm.at[b // 8, c0 : c0 + _HC, b % 8, :],
                buf.at[j, :, 0, :],
                sgx.at[j],
            )
            gath[j] = (g1, g2)

        def start_scatter(u):
            b = base + u // _UPR
            c0 = (u % _UPR) * _HC
            j = u % _NBUF
            gath[j][0].wait()
            gath[j][1].wait()
            h = _HC // 2
            s1 = pltpu.async_copy(
                buf.at[j, 0:h], out4_hbm.at[b, c0 : c0 + h, :, :], sss.at[j]
            )
            s2 = pltpu.async_copy(
                buf.at[j, h:_HC],
                out4_hbm.at[b, c0 + h : c0 + _HC, :, :],
                ss2.at[j],
            )
            scat[j] = (s1, s2)

        for u in range(_NU):
            j = u % _NBUF
            if u >= _NBUF:
                scat[j][0].wait()
                scat[j][1].wait()
            start_gather(u)
            if u >= 1:
                start_scatter(u - 1)
        start_scatter(_NU - 1)
        for j in range(_NBUF):
            scat[j][0].wait()
            scat[j][1].wait()

    out4 = push(x4, st4)
    return out4.transpose(0, 1, 3, 2).reshape(BATCH, NUM_NEURONS, HISTORY_LEN)


# speedup vs baseline: 1.0109x; 1.0109x over previous
import functools

import jax
import jax.numpy as jnp
from jax import lax
from jax.experimental import pallas as pl
from jax.experimental.pallas import tpu as pltpu
from jax.experimental.pallas import tpu_sc as plsc

BATCH = 512
NUM_NEURONS = 8192
HISTORY_LEN = 8
_NL = 128
_NC = NUM_NEURONS // _NL
_NW = 32
_RPW = BATCH // _NW


def kernel(x, state):
    st4 = state.reshape(BATCH, _NC, _NL, HISTORY_LEN).transpose(0, 1, 3, 2)
    x4 = x.reshape(BATCH // 8, 8, _NC, _NL).transpose(0, 2, 1, 3)

    mesh = plsc.VectorSubcoreMesh(core_axis_name="c", subcore_axis_name="s")

    _NBUF = 3
    _UPR = 2
    _HC = _NC // _UPR
    _NU = _RPW * _UPR

    @functools.partial(
        pl.kernel,
        mesh=mesh,
        out_type=jax.ShapeDtypeStruct((BATCH, _NC, HISTORY_LEN, _NL), jnp.float32),
        scratch_types=[
            pltpu.VMEM((_NBUF, _HC, HISTORY_LEN, _NL), jnp.float32),
            pltpu.SemaphoreType.DMA((_NBUF,)),
            pltpu.SemaphoreType.DMA((_NBUF,)),
            pltpu.SemaphoreType.DMA((_NBUF,)),
        ],
    )
    def push(x4_hbm, st4_hbm, out4_hbm, buf, sgs, sgx, sss):
        wid = lax.axis_index("s") * 2 + lax.axis_index("c")

        gath = [None] * _NBUF
        scat = [None] * _NBUF

        def start_gather(u):
            b = wid + _NW * (u // _UPR)
            c0 = (u % _UPR) * _HC
            j = u % _NBUF
            g1 = pltpu.async_copy(
                st4_hbm.at[b, c0 : c0 + _HC, 0 : HISTORY_LEN - 1, :],
                buf.at[j, :, 1:HISTORY_LEN, :],
                sgs.at[j],
            )
            g2 = pltpu.async_copy(
                x4_hbm.at[b // 8, c0 : c0 + _HC, b % 8, :],
                buf.at[j, :, 0, :],
                sgx.at[j],
            )
            gath[j] = (g1, g2)

        def start_scatter(u):
            b = wid + _NW * (u // _UPR)
            c0 = (u % _UPR) * _HC
            j = u % _NBUF
            gath[j][0].wait()
            gath[j][1].wait()
            scat[j] = pltpu.async_copy(
                buf.at[j], out4_hbm.at[b, c0 : c0 + _HC, :, :], sss.at[j]
            )

        for u in range(_NU):
            j = u % _NBUF
            if u >= _NBUF:
                scat[j].wait()
            start_gather(u)
            if u >= 1:
                start_scatter(u - 1)
        start_scatter(_NU - 1)
        for j in range(_NBUF):
            scat[j].wait()

    out4 = push(x4, st4)
    return out4.transpose(0, 1, 3, 2).reshape(BATCH, NUM_NEURONS, HISTORY_LEN)
